# baseline (speedup 1.0000x reference)
import functools

import jax
import jax.numpy as jnp
from jax import lax
from jax.experimental import pallas as pl
from jax.experimental.pallas import tpu as pltpu
from jax.experimental.pallas import tpu_sc as plsc

N_NODES = 10000
D_IN = 256
HALF = 128
N_EDGES = 160000

NC = 2
NS = 16
L = 16

EDGES_PER_TILE = N_EDGES // NS
CHUNK = 80
N_CHUNKS = EDGES_PER_TILE // CHUNK
NBUF = 4
ROWS_PER_TILE = 624
TAIL_BASE = ROWS_PER_TILE * NS
TAIL_ROWS = N_NODES - TAIL_BASE


def _sc_body(x2d, srcs1, dst1, w1, srcs2, dst2, w2, zeros, out1, out2,
             acc, src0, src1_, src2_, src3_, d0, d1, d2, d3,
             w0, w1_, w2_, w3_, rows0, rows1, rows2, rows3,
             g0, g1, g2, g3, s0, s1, s2, s3, i0, i1, i2, i3):
    c = lax.axis_index("c")
    s = lax.axis_index("s")
    row_base = s * ROWS_PER_TILE
    bufs = (rows0, rows1, rows2, rows3)
    srcset = (src0, src1_, src2_, src3_)
    dstset = (d0, d1, d2, d3)
    wset = (w0, w1_, w2_, w3_)
    gsem = (g0, g1, g2, g3)
    ssem = (s0, s1, s2, s3)
    isem = (i0, i1, i2, i3)

    def conv(srcs_h, dst_h, w_h, out_h):
        pltpu.sync_copy(zeros, acc.at[pl.ds(row_base, ROWS_PER_TILE)])

        @pl.when(s == NS - 1)
        def _zero_tail():
            pltpu.sync_copy(zeros.at[pl.ds(0, TAIL_ROWS)],
                            acc.at[pl.ds(TAIL_BASE, TAIL_ROWS)])

        e_base = s * EDGES_PER_TILE
        w_base = s * EDGES_PER_TILE * L

        def pf_src_d(i, b):
            return pltpu.make_async_copy(
                srcs_h.at[pl.ds(e_base + i * CHUNK, CHUNK)],
                srcset[b], isem[b])

        def pf_dst_d(i, b):
            return pltpu.make_async_copy(
                dst_h.at[pl.ds(e_base + i * CHUNK, CHUNK)],
                dstset[b], isem[b])

        def pf_w_d(i, b):
            return pltpu.make_async_copy(
                w_h.at[pl.ds(w_base + i * CHUNK * L, CHUNK * L)],
                wset[b], isem[b])

        def pf_start(i, b):
            pf_src_d(i, b).start()
            pf_dst_d(i, b).start()
            pf_w_d(i, b).start()

        def pf_wait(b):
            pf_src_d(0, b).wait()
            pf_dst_d(0, b).wait()
            pf_w_d(0, b).wait()

        def add_core_offset(b):
            ref = srcset[b]
            for k in range(CHUNK // L):
                sl = pl.ds(k * L, L)
                ref[sl] = ref[sl] + c

        def gather_d(b):
            return pltpu.make_async_copy(x2d.at[srcset[b]], bufs[b], gsem[b])

        def scatter_start(b):
            for k in range(CHUNK // L):
                idx = dstset[b][pl.ds(k * L, L)]
                pltpu.make_async_copy(
                    bufs[b].at[pl.ds(k * L, L)], acc.at[idx],
                    ssem[b]).start(add=True)

        def scatter_wait(b):
            pltpu.make_async_copy(x2d.at[pl.ds(0, CHUNK)], bufs[b],
                                  ssem[b]).wait()

        def scale(b):
            buf = bufs[b]
            wref = wset[b]

            def gbody(g, carry):
                for rr in range(L):
                    r = g * L + rr
                    wv = wref[pl.ds(r * L, L)]
                    for j in range(HALF // L):
                        fsl = pl.ds(j * L, L)
                        buf[r, fsl] = buf[r, fsl] * wv
                return carry

            lax.fori_loop(0, CHUNK // L, gbody, 0)

        pf_start(0, 0)
        pf_start(1, 1)
        plsc.subcore_barrier()
        pf_wait(0)
        add_core_offset(0)
        gather_d(0).start()

        def stage(j, b, do_prefetch):
            nb = (b + 1) % NBUF
            pf = (b + 2) % NBUF

            @pl.when(j >= NBUF - 1)
            def _drain():
                scatter_wait(nb)

            pf_wait(nb)
            add_core_offset(nb)
            gather_d(nb).start()
            if do_prefetch:
                pf_start(j + 2, pf)
            gather_d(b).wait()
            scale(b)
            scatter_start(b)

        def quad(p, carry):
            j = p * NBUF
            stage(j, 0, True)
            stage(j + 1, 1, True)
            stage(j + 2, 2, True)
            stage(j + 3, 3, True)
            return carry

        lax.fori_loop(0, 30, quad, 0)

        stage(120, 0, True)
        stage(121, 1, True)
        stage(122, 2, True)
        stage(123, 3, False)
        scatter_wait(1)
        gather_d(0).wait()
        scale(0)
        scatter_start(0)
        scatter_wait(2)
        scatter_wait(3)
        scatter_wait(0)

        plsc.subcore_barrier()
        pltpu.sync_copy(acc.at[pl.ds(row_base, ROWS_PER_TILE)],
                        out_h.at[pl.ds(c * N_NODES + row_base, ROWS_PER_TILE)])

        @pl.when(s == NS - 1)
        def _write_tail():
            pltpu.sync_copy(acc.at[pl.ds(TAIL_BASE, TAIL_ROWS)],
                            out_h.at[pl.ds(c * N_NODES + TAIL_BASE, TAIL_ROWS)])

        plsc.subcore_barrier()

    conv(srcs1, dst1, w1, out1)
    conv(srcs2, dst2, w2, out2)


_sc_conv = functools.partial(
    pl.kernel,
    out_type=(
        jax.ShapeDtypeStruct((NC * N_NODES, HALF), jnp.float32),
        jax.ShapeDtypeStruct((NC * N_NODES, HALF), jnp.float32),
    ),
    mesh=plsc.VectorSubcoreMesh(core_axis_name="c", subcore_axis_name="s"),
    scratch_types=(
        [pltpu.VMEM_SHARED((N_NODES, HALF), jnp.float32)]
        + [pltpu.VMEM((CHUNK,), jnp.int32) for _ in range(NBUF)]
        + [pltpu.VMEM((CHUNK,), jnp.int32) for _ in range(NBUF)]
        + [pltpu.VMEM((CHUNK * L,), jnp.float32) for _ in range(NBUF)]
        + [pltpu.VMEM((CHUNK, HALF), jnp.float32) for _ in range(NBUF)]
        + [pltpu.SemaphoreType.DMA for _ in range(3 * NBUF)]
    ),
)(_sc_body)


BLK = 1000


def _tc_x0_body(x_b, wl, bl, o0):
    o0[...] = (jnp.dot(x_b[...], wl[...], preferred_element_type=jnp.float32)
               + bl[...])


def _tc_x0(x, wl, bl):
    n_blk = N_NODES // BLK
    return pl.pallas_call(
        _tc_x0_body,
        grid=(n_blk,),
        in_specs=[
            pl.BlockSpec((BLK, D_IN), lambda i: (i, 0)),
            pl.BlockSpec((D_IN, D_IN), lambda i: (0, 0)),
            pl.BlockSpec((1, D_IN), lambda i: (0, 0)),
        ],
        out_specs=pl.BlockSpec((BLK, D_IN), lambda i: (i, 0)),
        out_shape=jax.ShapeDtypeStruct((N_NODES, D_IN), jnp.float32),
    )(x, wl, bl)


def _tc_body(a1lo, a1hi, a2lo, a2hi, w1a, w1b, w2a, w2b, b1, b2, o1, o2):
    f32 = jnp.float32
    o1[...] = (jnp.dot(a1lo[...], w1a[...], preferred_element_type=f32)
               + jnp.dot(a1hi[...], w1b[...], preferred_element_type=f32)
               + b1[...])
    o2[...] = (jnp.dot(a2lo[...], w2a[...], preferred_element_type=f32)
               + jnp.dot(a2hi[...], w2b[...], preferred_element_type=f32)
               + b2[...])


def _tc_matmuls(agg1f, agg2f, w1a, w1b, w2a, w2b, b1, b2):
    n_blk = N_NODES // BLK
    lo_spec = pl.BlockSpec((BLK, HALF), lambda i: (i, 0))
    hi_spec = pl.BlockSpec((BLK, HALF), lambda i: (i + n_blk, 0))
    wh_spec = pl.BlockSpec((HALF, D_IN), lambda i: (0, 0))
    b_spec = pl.BlockSpec((1, D_IN), lambda i: (0, 0))
    out_spec = pl.BlockSpec((BLK, D_IN), lambda i: (i, 0))
    out_shape = jax.ShapeDtypeStruct((N_NODES, D_IN), jnp.float32)
    return pl.pallas_call(
        _tc_body,
        grid=(n_blk,),
        in_specs=[
            lo_spec, hi_spec, lo_spec, hi_spec,
            wh_spec, wh_spec, wh_spec, wh_spec,
            b_spec, b_spec,
        ],
        out_specs=[out_spec, out_spec],
        out_shape=[out_shape, out_shape],
    )(agg1f, agg1f, agg2f, agg2f, w1a, w1b, w2a, w2b, b1, b2)


def kernel(x, edge_index, edge_weight, edge_index2, edge_weight2,
           W_ln, b_ln, W1, b1, W2, b2):
    x = x.astype(jnp.float32)
    x2d = x.reshape(NC * N_NODES, HALF)

    def prep(edge_index):
        src2 = 2 * edge_index[0].astype(jnp.int32)
        dst = edge_index[1].astype(jnp.int32)
        return src2, dst

    srcs1, dst1 = prep(edge_index)
    srcs2, dst2 = prep(edge_index2)

    def wexp(w):
        w = w.astype(jnp.float32)
        return jnp.broadcast_to(w[:, None], (N_EDGES, L)).reshape(-1)

    w1r = wexp(edge_weight)
    w2r = wexp(edge_weight2)
    zeros = jnp.zeros((ROWS_PER_TILE, HALF), jnp.float32)

    agg1f, agg2f = _sc_conv(x2d, srcs1, dst1, w1r, srcs2, dst2, w2r, zeros)

    x0 = _tc_x0(x, W_ln.T, b_ln.reshape(1, D_IN))
    x1, x2 = _tc_matmuls(
        agg1f, agg2f, W1[:HALF], W1[HALF:], W2[:HALF], W2[HALF:],
        b1.reshape(1, D_IN), b2.reshape(1, D_IN))
    return (x0, x1, x2)


# speedup vs baseline: 1.8407x; 1.8407x over previous
import functools

import jax
import jax.numpy as jnp
from jax import lax
from jax.experimental import pallas as pl
from jax.experimental.pallas import tpu as pltpu
from jax.experimental.pallas import tpu_sc as plsc

N_NODES = 10000
D_IN = 256
HALF = 128
N_EDGES = 160000

NC = 2
NS = 16
L = 16

EDGES_PER_TILE = N_EDGES // NS
CHUNK = 80
N_CHUNKS = EDGES_PER_TILE // CHUNK
NBUF = 3
ROWS_PER_TILE = 624
TAIL_BASE = ROWS_PER_TILE * NS
TAIL_ROWS = N_NODES - TAIL_BASE


def _sc_body(x2d, srcs1, dst1, w1, srcs2, dst2, w2, zeros, out1, out2,
             acc, dst_all, src0, src1_, src2_, w0, w1_, w2_,
             rows0, rows1, rows2,
             g0, g1, g2, s0, s1, s2, i0, i1, i2):
    c = lax.axis_index("c")
    s = lax.axis_index("s")
    row_base = s * ROWS_PER_TILE
    bufs = (rows0, rows1, rows2)
    srcset = (src0, src1_, src2_)
    wset = (w0, w1_, w2_)
    gsem = (g0, g1, g2)
    ssem = (s0, s1, s2)
    isem = (i0, i1, i2)

    def conv(srcs_h, dst_h, w_h, out_h):
        pltpu.sync_copy(dst_h.at[s], dst_all)
        pltpu.sync_copy(zeros, acc.at[pl.ds(row_base, ROWS_PER_TILE)])

        @pl.when(s == NS - 1)
        def _zero_tail():
            pltpu.sync_copy(zeros.at[pl.ds(0, TAIL_ROWS)],
                            acc.at[pl.ds(TAIL_BASE, TAIL_ROWS)])

        src_base = s * EDGES_PER_TILE
        w_base = s * EDGES_PER_TILE

        def pf_src_d(i, b):
            return pltpu.make_async_copy(
                srcs_h.at[pl.ds(src_base + i * CHUNK, CHUNK)],
                srcset[b], isem[b])

        def pf_w_d(i, b):
            return pltpu.make_async_copy(
                w_h.at[pl.ds(w_base + i * CHUNK, CHUNK)], wset[b], isem[b])

        def add_core_offset(b):
            ref = srcset[b]
            for k in range(CHUNK // L):
                sl = pl.ds(k * L, L)
                ref[sl] = ref[sl] + c

        def gather_d(b):
            return pltpu.make_async_copy(x2d.at[srcset[b]], bufs[b], gsem[b])

        def scatter_d(i, b):
            return pltpu.make_async_copy(bufs[b], acc.at[dst_all.at[i]],
                                         ssem[b])

        def scale(b):
            buf = bufs[b]
            wref = wset[b]

            def gbody(g, carry):
                wvec = wref[pl.ds(g * L, L)]
                for rr in range(L):
                    r = g * L + rr
                    wv = wvec[rr]
                    for j in range(HALF // L):
                        fsl = pl.ds(j * L, L)
                        buf[r, fsl] = buf[r, fsl] * wv
                return carry

            lax.fori_loop(0, CHUNK // L, gbody, 0)

        pf_src_d(0, 0).start()
        pf_w_d(0, 0).start()
        pf_src_d(1, 1).start()
        pf_w_d(1, 1).start()
        plsc.subcore_barrier()
        pf_src_d(0, 0).wait()
        pf_w_d(0, 0).wait()
        add_core_offset(0)
        gather_d(0).start()

        def stage(j, b, do_prefetch):
            nb = (b + 1) % NBUF
            pf = (b + 2) % NBUF

            @pl.when(j >= 2)
            def _drain():
                scatter_d(0, nb).wait()

            pf_src_d(0, nb).wait()
            pf_w_d(0, nb).wait()
            add_core_offset(nb)
            gather_d(nb).start()
            if do_prefetch:
                pf_src_d(j + 2, pf).start()
                pf_w_d(j + 2, pf).start()
            gather_d(b).wait()
            scale(b)
            scatter_d(j, b).start(add=True)

        def triad(p, carry):
            j = p * NBUF
            stage(j, 0, True)
            stage(j + 1, 1, True)
            stage(j + 2, 2, True)
            return carry

        lax.fori_loop(0, (N_CHUNKS - 2) // NBUF, triad, 0)

        j = N_CHUNKS - 2
        scatter_d(0, 1).wait()
        pf_src_d(0, 1).wait()
        pf_w_d(0, 1).wait()
        add_core_offset(1)
        gather_d(1).start()
        gather_d(0).wait()
        scale(0)
        scatter_d(j, 0).start(add=True)
        scatter_d(0, 2).wait()
        gather_d(1).wait()
        scale(1)
        scatter_d(j + 1, 1).start(add=True)
        scatter_d(0, 0).wait()
        scatter_d(0, 1).wait()

        plsc.subcore_barrier()
        pltpu.sync_copy(acc.at[pl.ds(row_base, ROWS_PER_TILE)],
                        out_h.at[pl.ds(c * N_NODES + row_base, ROWS_PER_TILE)])

        @pl.when(s == NS - 1)
        def _write_tail():
            pltpu.sync_copy(acc.at[pl.ds(TAIL_BASE, TAIL_ROWS)],
                            out_h.at[pl.ds(c * N_NODES + TAIL_BASE, TAIL_ROWS)])

        plsc.subcore_barrier()

    conv(srcs1, dst1, w1, out1)
    conv(srcs2, dst2, w2, out2)


_sc_conv = functools.partial(
    pl.kernel,
    out_type=(
        jax.ShapeDtypeStruct((NC * N_NODES, HALF), jnp.float32),
        jax.ShapeDtypeStruct((NC * N_NODES, HALF), jnp.float32),
    ),
    mesh=plsc.VectorSubcoreMesh(core_axis_name="c", subcore_axis_name="s"),
    scratch_types=[
        pltpu.VMEM_SHARED((N_NODES, HALF), jnp.float32),
        pltpu.VMEM((N_CHUNKS, CHUNK), jnp.int32),
        pltpu.VMEM((CHUNK,), jnp.int32),
        pltpu.VMEM((CHUNK,), jnp.int32),
        pltpu.VMEM((CHUNK,), jnp.int32),
        pltpu.VMEM((CHUNK,), jnp.float32),
        pltpu.VMEM((CHUNK,), jnp.float32),
        pltpu.VMEM((CHUNK,), jnp.float32),
        pltpu.VMEM((CHUNK, HALF), jnp.float32),
        pltpu.VMEM((CHUNK, HALF), jnp.float32),
        pltpu.VMEM((CHUNK, HALF), jnp.float32),
        pltpu.SemaphoreType.DMA,
        pltpu.SemaphoreType.DMA,
        pltpu.SemaphoreType.DMA,
        pltpu.SemaphoreType.DMA,
        pltpu.SemaphoreType.DMA,
        pltpu.SemaphoreType.DMA,
        pltpu.SemaphoreType.DMA,
        pltpu.SemaphoreType.DMA,
        pltpu.SemaphoreType.DMA,
    ],
)(_sc_body)


BLK = 1000


def _tc_x0_body(x_b, wl, bl, o0):
    o0[...] = (jnp.dot(x_b[...], wl[...], preferred_element_type=jnp.float32)
               + bl[...])


def _tc_x0(x, wl, bl):
    n_blk = N_NODES // BLK
    return pl.pallas_call(
        _tc_x0_body,
        grid=(n_blk,),
        in_specs=[
            pl.BlockSpec((BLK, D_IN), lambda i: (i, 0)),
            pl.BlockSpec((D_IN, D_IN), lambda i: (0, 0)),
            pl.BlockSpec((1, D_IN), lambda i: (0, 0)),
        ],
        out_specs=pl.BlockSpec((BLK, D_IN), lambda i: (i, 0)),
        out_shape=jax.ShapeDtypeStruct((N_NODES, D_IN), jnp.float32),
    )(x, wl, bl)


def _tc_body(a1lo, a1hi, a2lo, a2hi, w1a, w1b, w2a, w2b, b1, b2, o1, o2):
    f32 = jnp.float32
    o1[...] = (jnp.dot(a1lo[...], w1a[...], preferred_element_type=f32)
               + jnp.dot(a1hi[...], w1b[...], preferred_element_type=f32)
               + b1[...])
    o2[...] = (jnp.dot(a2lo[...], w2a[...], preferred_element_type=f32)
               + jnp.dot(a2hi[...], w2b[...], preferred_element_type=f32)
               + b2[...])


def _tc_matmuls(agg1f, agg2f, w1a, w1b, w2a, w2b, b1, b2):
    n_blk = N_NODES // BLK
    lo_spec = pl.BlockSpec((BLK, HALF), lambda i: (i, 0))
    hi_spec = pl.BlockSpec((BLK, HALF), lambda i: (i + n_blk, 0))
    wh_spec = pl.BlockSpec((HALF, D_IN), lambda i: (0, 0))
    b_spec = pl.BlockSpec((1, D_IN), lambda i: (0, 0))
    out_spec = pl.BlockSpec((BLK, D_IN), lambda i: (i, 0))
    out_shape = jax.ShapeDtypeStruct((N_NODES, D_IN), jnp.float32)
    return pl.pallas_call(
        _tc_body,
        grid=(n_blk,),
        in_specs=[
            lo_spec, hi_spec, lo_spec, hi_spec,
            wh_spec, wh_spec, wh_spec, wh_spec,
            b_spec, b_spec,
        ],
        out_specs=[out_spec, out_spec],
        out_shape=[out_shape, out_shape],
    )(agg1f, agg1f, agg2f, agg2f, w1a, w1b, w2a, w2b, b1, b2)


def kernel(x, edge_index, edge_weight, edge_index2, edge_weight2,
           W_ln, b_ln, W1, b1, W2, b2):
    x = x.astype(jnp.float32)
    x2d = x.reshape(NC * N_NODES, HALF)

    def prep(edge_index):
        src2 = 2 * edge_index[0].astype(jnp.int32)
        dst = edge_index[1].astype(jnp.int32)
        return src2, dst.reshape(NS, N_CHUNKS, CHUNK)

    srcs1, dst1 = prep(edge_index)
    srcs2, dst2 = prep(edge_index2)
    w1r = edge_weight.astype(jnp.float32)
    w2r = edge_weight2.astype(jnp.float32)
    zeros = jnp.zeros((ROWS_PER_TILE, HALF), jnp.float32)

    agg1f, agg2f = _sc_conv(x2d, srcs1, dst1, w1r, srcs2, dst2, w2r, zeros)

    x0 = _tc_x0(x, W_ln.T, b_ln.reshape(1, D_IN))
    x1, x2 = _tc_matmuls(
        agg1f, agg2f, W1[:HALF], W1[HALF:], W2[:HALF], W2[HALF:],
        b1.reshape(1, D_IN), b2.reshape(1, D_IN))
    return (x0, x1, x2)


# speedup vs baseline: 1.8761x; 1.0192x over previous
import functools

import jax
import jax.numpy as jnp
from jax import lax
from jax.experimental import pallas as pl
from jax.experimental.pallas import tpu as pltpu
from jax.experimental.pallas import tpu_sc as plsc

N_NODES = 10000
D_IN = 256
HALF = 128
N_EDGES = 160000

NC = 2
NS = 16
L = 16

EDGES_PER_TILE = N_EDGES // NS
CHUNK = 80
N_CHUNKS = EDGES_PER_TILE // CHUNK
NBUF = 3
ROWS_PER_TILE = 624
TAIL_BASE = ROWS_PER_TILE * NS
TAIL_ROWS = N_NODES - TAIL_BASE


def _sc_body(x2d, srcs1, dst1, w1, srcs2, dst2, w2, zeros, out1, out2,
             acc, dst_all, src0, src1_, src2_, w0, w1_, w2_,
             rows0, rows1, rows2,
             g0, g1, g2, s0, s1, s2, i0, i1, i2):
    c = lax.axis_index("c")
    s = lax.axis_index("s")
    row_base = s * ROWS_PER_TILE
    bufs = (rows0, rows1, rows2)
    srcset = (src0, src1_, src2_)
    wset = (w0, w1_, w2_)
    gsem = (g0, g1, g2)
    ssem = (s0, s1, s2)
    isem = (i0, i1, i2)

    def conv(srcs_h, dst_h, w_h, out_h):
        pltpu.sync_copy(zeros, acc.at[pl.ds(row_base, ROWS_PER_TILE)])

        @pl.when(s == NS - 1)
        def _zero_tail():
            pltpu.sync_copy(zeros.at[pl.ds(0, TAIL_ROWS)],
                            acc.at[pl.ds(TAIL_BASE, TAIL_ROWS)])

        src_base = s * EDGES_PER_TILE
        w_base = s * EDGES_PER_TILE

        def pf_src_d(i, b):
            return pltpu.make_async_copy(
                srcs_h.at[pl.ds(src_base + i * CHUNK, CHUNK)],
                srcset[b], isem[b])

        def pf_dst_d(i, b):
            return pltpu.make_async_copy(
                dst_h.at[pl.ds(src_base + i * CHUNK, CHUNK)],
                dst_all.at[pl.ds(i * CHUNK, CHUNK)], isem[b])

        def pf_w_d(i, b):
            return pltpu.make_async_copy(
                w_h.at[pl.ds(w_base + i * CHUNK, CHUNK)], wset[b], isem[b])

        def add_core_offset(b):
            ref = srcset[b]
            for k in range(CHUNK // L):
                sl = pl.ds(k * L, L)
                ref[sl] = ref[sl] + c

        def gather_d(b):
            return pltpu.make_async_copy(x2d.at[srcset[b]], bufs[b], gsem[b])

        def scatter_d(i, b):
            return pltpu.make_async_copy(
                bufs[b], acc.at[dst_all.at[pl.ds(i * CHUNK, CHUNK)]],
                ssem[b])

        def scale(b):
            buf = bufs[b]
            wref = wset[b]

            def gbody(g, carry):
                wvec = wref[pl.ds(g * L, L)]
                for rr in range(L):
                    r = g * L + rr
                    wv = wvec[rr]
                    for j in range(HALF // L):
                        fsl = pl.ds(j * L, L)
                        buf[r, fsl] = buf[r, fsl] * wv
                return carry

            lax.fori_loop(0, CHUNK // L, gbody, 0)

        pf_src_d(0, 0).start()
        pf_dst_d(0, 0).start()
        pf_w_d(0, 0).start()
        pf_src_d(1, 1).start()
        pf_dst_d(1, 1).start()
        pf_w_d(1, 1).start()
        plsc.subcore_barrier()
        pf_src_d(0, 0).wait()
        pf_dst_d(0, 0).wait()
        pf_w_d(0, 0).wait()
        add_core_offset(0)
        gather_d(0).start()

        def stage(j, b, do_prefetch):
            nb = (b + 1) % NBUF
            pf = (b + 2) % NBUF

            @pl.when(j >= 2)
            def _drain():
                scatter_d(0, nb).wait()

            pf_src_d(0, nb).wait()
            pf_dst_d(0, nb).wait()
            pf_w_d(0, nb).wait()
            add_core_offset(nb)
            gather_d(nb).start()
            if do_prefetch:
                pf_src_d(j + 2, pf).start()
                pf_dst_d(j + 2, pf).start()
                pf_w_d(j + 2, pf).start()
            gather_d(b).wait()
            scale(b)
            scatter_d(j, b).start(add=True)

        def triad(p, carry):
            j = p * NBUF
            stage(j, 0, True)
            stage(j + 1, 1, True)
            stage(j + 2, 2, True)
            return carry

        lax.fori_loop(0, (N_CHUNKS - 2) // NBUF, triad, 0)

        j = N_CHUNKS - 2
        scatter_d(0, 1).wait()
        pf_src_d(0, 1).wait()
        pf_dst_d(0, 1).wait()
        pf_w_d(0, 1).wait()
        add_core_offset(1)
        gather_d(1).start()
        gather_d(0).wait()
        scale(0)
        scatter_d(j, 0).start(add=True)
        scatter_d(0, 2).wait()
        gather_d(1).wait()
        scale(1)
        scatter_d(j + 1, 1).start(add=True)
        scatter_d(0, 0).wait()
        scatter_d(0, 1).wait()

        plsc.subcore_barrier()
        pltpu.sync_copy(acc.at[pl.ds(row_base, ROWS_PER_TILE)],
                        out_h.at[pl.ds(c * N_NODES + row_base, ROWS_PER_TILE)])

        @pl.when(s == NS - 1)
        def _write_tail():
            pltpu.sync_copy(acc.at[pl.ds(TAIL_BASE, TAIL_ROWS)],
                            out_h.at[pl.ds(c * N_NODES + TAIL_BASE, TAIL_ROWS)])

        plsc.subcore_barrier()

    conv(srcs1, dst1, w1, out1)
    conv(srcs2, dst2, w2, out2)


_sc_conv = functools.partial(
    pl.kernel,
    out_type=(
        jax.ShapeDtypeStruct((NC * N_NODES, HALF), jnp.float32),
        jax.ShapeDtypeStruct((NC * N_NODES, HALF), jnp.float32),
    ),
    mesh=plsc.VectorSubcoreMesh(core_axis_name="c", subcore_axis_name="s"),
    scratch_types=[
        pltpu.VMEM_SHARED((N_NODES, HALF), jnp.float32),
        pltpu.VMEM((EDGES_PER_TILE,), jnp.int32),
        pltpu.VMEM((CHUNK,), jnp.int32),
        pltpu.VMEM((CHUNK,), jnp.int32),
        pltpu.VMEM((CHUNK,), jnp.int32),
        pltpu.VMEM((CHUNK,), jnp.float32),
        pltpu.VMEM((CHUNK,), jnp.float32),
        pltpu.VMEM((CHUNK,), jnp.float32),
        pltpu.VMEM((CHUNK, HALF), jnp.float32),
        pltpu.VMEM((CHUNK, HALF), jnp.float32),
        pltpu.VMEM((CHUNK, HALF), jnp.float32),
        pltpu.SemaphoreType.DMA,
        pltpu.SemaphoreType.DMA,
        pltpu.SemaphoreType.DMA,
        pltpu.SemaphoreType.DMA,
        pltpu.SemaphoreType.DMA,
        pltpu.SemaphoreType.DMA,
        pltpu.SemaphoreType.DMA,
        pltpu.SemaphoreType.DMA,
        pltpu.SemaphoreType.DMA,
    ],
)(_sc_body)


BLK = 1000


def _tc_x0_body(x_b, wl, bl, o0):
    o0[...] = (jnp.dot(x_b[...], wl[...], preferred_element_type=jnp.float32)
               + bl[...])


def _tc_x0(x, wl, bl):
    n_blk = N_NODES // BLK
    return pl.pallas_call(
        _tc_x0_body,
        grid=(n_blk,),
        in_specs=[
            pl.BlockSpec((BLK, D_IN), lambda i: (i, 0)),
            pl.BlockSpec((D_IN, D_IN), lambda i: (0, 0)),
            pl.BlockSpec((1, D_IN), lambda i: (0, 0)),
        ],
        out_specs=pl.BlockSpec((BLK, D_IN), lambda i: (i, 0)),
        out_shape=jax.ShapeDtypeStruct((N_NODES, D_IN), jnp.float32),
    )(x, wl, bl)


def _tc_body(a1lo, a1hi, a2lo, a2hi, w1a, w1b, w2a, w2b, b1, b2, o1, o2):
    f32 = jnp.float32
    o1[...] = (jnp.dot(a1lo[...], w1a[...], preferred_element_type=f32)
               + jnp.dot(a1hi[...], w1b[...], preferred_element_type=f32)
               + b1[...])
    o2[...] = (jnp.dot(a2lo[...], w2a[...], preferred_element_type=f32)
               + jnp.dot(a2hi[...], w2b[...], preferred_element_type=f32)
               + b2[...])


def _tc_matmuls(agg1f, agg2f, w1a, w1b, w2a, w2b, b1, b2):
    n_blk = N_NODES // BLK
    lo_spec = pl.BlockSpec((BLK, HALF), lambda i: (i, 0))
    hi_spec = pl.BlockSpec((BLK, HALF), lambda i: (i + n_blk, 0))
    wh_spec = pl.BlockSpec((HALF, D_IN), lambda i: (0, 0))
    b_spec = pl.BlockSpec((1, D_IN), lambda i: (0, 0))
    out_spec = pl.BlockSpec((BLK, D_IN), lambda i: (i, 0))
    out_shape = jax.ShapeDtypeStruct((N_NODES, D_IN), jnp.float32)
    return pl.pallas_call(
        _tc_body,
        grid=(n_blk,),
        in_specs=[
            lo_spec, hi_spec, lo_spec, hi_spec,
            wh_spec, wh_spec, wh_spec, wh_spec,
            b_spec, b_spec,
        ],
        out_specs=[out_spec, out_spec],
        out_shape=[out_shape, out_shape],
    )(agg1f, agg1f, agg2f, agg2f, w1a, w1b, w2a, w2b, b1, b2)


def kernel(x, edge_index, edge_weight, edge_index2, edge_weight2,
           W_ln, b_ln, W1, b1, W2, b2):
    x = x.astype(jnp.float32)
    x2d = x.reshape(NC * N_NODES, HALF)

    def prep(edge_index):
        src2 = 2 * edge_index[0].astype(jnp.int32)
        dst = edge_index[1].astype(jnp.int32)
        return src2, dst

    srcs1, dst1 = prep(edge_index)
    srcs2, dst2 = prep(edge_index2)
    w1r = edge_weight.astype(jnp.float32)
    w2r = edge_weight2.astype(jnp.float32)
    zeros = jnp.zeros((ROWS_PER_TILE, HALF), jnp.float32)

    agg1f, agg2f = _sc_conv(x2d, srcs1, dst1, w1r, srcs2, dst2, w2r, zeros)

    x0 = _tc_x0(x, W_ln.T, b_ln.reshape(1, D_IN))
    x1, x2 = _tc_matmuls(
        agg1f, agg2f, W1[:HALF], W1[HALF:], W2[:HALF], W2[HALF:],
        b1.reshape(1, D_IN), b2.reshape(1, D_IN))
    return (x0, x1, x2)
